# baseline (speedup 1.0000x reference)
import functools

import jax
import jax.numpy as jnp
from jax import lax
from jax.experimental import pallas as pl
from jax.experimental.pallas import tpu as pltpu
from jax.experimental.pallas import tpu_sc as plsc

NB_USER = 1000000
NB_ITEM = 1000000
F = 32
B = 16384

NC, NS, L = 2, 16, 16
NW = NC * NS
BPW = B // NW


def _mf_body(users_hbm, items_hbm, ue_hbm, ie_hbm, ub_hbm, ib_hbm, gb_hbm,
             out_hbm,
             uidx, iidx, urows, irows, ubias, ibias, gbv, ob,
             s0, s1, s2, s3):
    wid = lax.axis_index("s") * NC + lax.axis_index("c")
    base = wid * BPW

    pltpu.sync_copy(users_hbm.at[pl.ds(base, BPW)], uidx)
    pltpu.sync_copy(items_hbm.at[pl.ds(base, BPW)], iidx)

    cu = pltpu.async_copy(ue_hbm.at[uidx], urows, s0)
    ci = pltpu.async_copy(ie_hbm.at[iidx], irows, s1)
    cub = pltpu.async_copy(ub_hbm.at[uidx], ubias, s2)
    cib = pltpu.async_copy(ib_hbm.at[iidx], ibias, s3)
    pltpu.sync_copy(gb_hbm, gbv.at[pl.ds(0, 1)])
    cu.wait()
    ci.wait()
    cub.wait()
    cib.wait()

    gb = gbv[...][0]
    lane = lax.iota(jnp.int32, L)

    def group(g, carry):
        rows = lane + g * L
        acc = jnp.zeros((L,), jnp.float32)
        for f in range(F):
            col = jnp.full((L,), f, jnp.int32)
            acc = acc + (plsc.load_gather(urows, [rows, col])
                         * plsc.load_gather(irows, [rows, col]))
        acc = acc + ubias[pl.ds(g * L, L)]
        acc = acc + ibias[pl.ds(g * L, L)]
        ob[pl.ds(g * L, L)] = acc + gb
        return carry

    lax.fori_loop(0, BPW // L, group, 0)
    pltpu.sync_copy(ob, out_hbm.at[pl.ds(base, BPW)])


@jax.jit
def _mf(users, items, user_embeddings, item_embeddings, user_biases,
        item_biases, global_bias):
    mesh = plsc.VectorSubcoreMesh(core_axis_name="c", subcore_axis_name="s")
    run = pl.kernel(
        _mf_body,
        out_type=jax.ShapeDtypeStruct((B,), jnp.float32),
        mesh=mesh,
        compiler_params=pltpu.CompilerParams(
            needs_layout_passes=False, use_tc_tiling_on_sc=False),
        scratch_types=[
            pltpu.VMEM((BPW,), jnp.int32),
            pltpu.VMEM((BPW,), jnp.int32),
            pltpu.VMEM((BPW, F), jnp.float32),
            pltpu.VMEM((BPW, F), jnp.float32),
            pltpu.VMEM((BPW,), jnp.float32),
            pltpu.VMEM((BPW,), jnp.float32),
            pltpu.VMEM((L,), jnp.float32),
            pltpu.VMEM((BPW,), jnp.float32),
            pltpu.SemaphoreType.DMA,
            pltpu.SemaphoreType.DMA,
            pltpu.SemaphoreType.DMA,
            pltpu.SemaphoreType.DMA,
        ],
    )
    out = run(users, items, user_embeddings, item_embeddings,
              user_biases.reshape(NB_USER), item_biases.reshape(NB_ITEM),
              global_bias)
    return out.reshape(B, 1)


def kernel(users, items, user_embeddings, item_embeddings, user_biases,
           item_biases, global_bias):
    return _mf(users.astype(jnp.int32), items.astype(jnp.int32),
               user_embeddings, item_embeddings, user_biases, item_biases,
               global_bias)
